# baseline (speedup 1.0000x reference)
import functools
import math

import jax
import jax.numpy as jnp
from jax import lax
from jax.experimental import pallas as pl
from jax.experimental.pallas import tpu as pltpu
from jax.experimental.pallas import tpu_sc as plsc

_L = 16
_LOG01 = math.log(0.1)
_NEG = float("-inf")

_B = 64
_V = 1000000
_V_SC = 100800
_V_TC = _V - _V_SC
_TC_CH = 3200
_TC_NB = _V_TC // _TC_CH


def _insert4(t0, t1, t2, t3, x):
  m0 = jnp.maximum(t0, x)
  w = jnp.minimum(t0, x)
  m1 = jnp.maximum(t1, w)
  w = jnp.minimum(t1, w)
  m2 = jnp.maximum(t2, w)
  w = jnp.minimum(t2, w)
  m3 = jnp.maximum(t3, w)
  return m0, m1, m2, m3


def _merge4(a, b):
  a0, a1, a2, a3 = a
  b0, b1, b2, b3 = b
  h0 = jnp.maximum(a0, b3)
  h1 = jnp.maximum(a1, b2)
  h2 = jnp.maximum(a2, b1)
  h3 = jnp.maximum(a3, b0)
  p0 = jnp.maximum(h0, h2)
  p2 = jnp.minimum(h0, h2)
  p1 = jnp.maximum(h1, h3)
  p3 = jnp.minimum(h1, h3)
  t0 = jnp.maximum(p0, p1)
  t1 = jnp.minimum(p0, p1)
  t2 = jnp.maximum(p2, p3)
  t3 = jnp.minimum(p2, p3)
  return t0, t1, t2, t3


def _build_sc(vocab, batch, base_off, width, n_chunks, unroll, rep_unroll,
              nbuf):
  chunk = width // n_chunks
  assert chunk * n_chunks == width and chunk % (_L * unroll) == 0
  vecs = chunk // _L
  inner = vecs // unroll
  rep_inner = vecs // rep_unroll
  assert rep_inner * rep_unroll == vecs
  assert n_chunks % nbuf == 0 and chunk % 8 == 0 and base_off % 8 == 0

  info = plsc.get_sparse_core_info()
  num_cores, num_subcores = info.num_cores, info.num_subcores
  nw = num_cores * num_subcores
  rows_per_tile = batch // nw
  assert rows_per_tile * nw == batch

  mesh = plsc.VectorSubcoreMesh(
      core_axis_name="c", subcore_axis_name="s",
      num_cores=num_cores, num_subcores=num_subcores)

  @functools.partial(
      pl.kernel,
      out_type=jax.ShapeDtypeStruct((batch, _L), jnp.float32),
      mesh=mesh,
      scratch_types=(
          [pltpu.VMEM((chunk,), jnp.float32) for _ in range(nbuf)]
          + [
              pltpu.VMEM((_L,), jnp.float32),
              pltpu.VMEM((_L,), jnp.float32),
              pltpu.SMEM((n_chunks,), jnp.float32),
              pltpu.SMEM((n_chunks,), jnp.int32),
          ]
          + [pltpu.SemaphoreType.DMA for _ in range(nbuf)]
      ),
      compiler_params=pltpu.CompilerParams(needs_layout_passes=False),
  )
  def sc_kernel(x_hbm, out_hbm, *scratch):
    bufs = scratch[:nbuf]
    head, outb, cmax, cand = scratch[nbuf:nbuf + 4]
    sems = scratch[nbuf + 4:]
    wid = lax.axis_index("s") * num_cores + lax.axis_index("c")
    neg_v = jnp.full((_L,), _NEG, dtype=jnp.float32)
    ii = lax.iota(jnp.int32, _L)

    for r in range(rows_per_tile):
      row = wid * rows_per_tile + r
      base = row * vocab + base_off

      pltpu.sync_copy(x_hbm.at[pl.ds(base, _L)], head)
      mv = jnp.broadcast_to(jnp.max(head[...]), (_L,))

      for p in range(nbuf):
        pltpu.async_copy(x_hbm.at[pl.ds(base + p * chunk, chunk)], bufs[p],
                         sems[p])

      def chunk_step(c, p, carry):
        mvc, svs = carry
        src = x_hbm.at[pl.ds(base + c * chunk, chunk)]
        pltpu.make_async_copy(src, bufs[p], sems[p]).wait()

        def inner_body(j, car):
          cms, sv2 = list(car[0]), list(car[1])
          off = j * (unroll * _L)
          for u in range(unroll):
            bk = u % 5
            x = bufs[p][pl.ds(off + u * _L, _L)]
            cms[bk] = jnp.maximum(cms[bk], x)
            sv2[bk] = sv2[bk] + jnp.exp(x - mvc)
          return tuple(cms), tuple(sv2)

        cms0 = (neg_v,) * 5
        cms, svs = lax.fori_loop(0, inner, inner_body, (cms0, svs),
                                 unroll=False)

        @pl.when(c + nbuf < n_chunks)
        def _():
          nxt = x_hbm.at[pl.ds(base + (c + nbuf) * chunk, chunk)]
          pltpu.async_copy(nxt, bufs[p], sems[p])

        cm = jnp.maximum(jnp.maximum(cms[0], cms[1]),
                         jnp.maximum(jnp.maximum(cms[2], cms[3]), cms[4]))
        mc = jnp.max(cm)
        cmax[c] = mc
        mvn = jnp.maximum(mvc, jnp.broadcast_to(mc, (_L,)))
        scale = jnp.exp(mvc - mvn)
        svs = tuple(s * scale for s in svs)
        return mvn, svs

      def outer_body(i, carry):
        for p in range(nbuf):
          carry = chunk_step(nbuf * i + p, p, carry)
        return carry

      zeros = jnp.zeros((_L,), jnp.float32)
      mv, svs = lax.fori_loop(0, n_chunks // nbuf, outer_body,
                              (mv, (zeros,) * 5))
      sv = ((svs[0] + svs[1]) + (svs[2] + svs[3])) + svs[4]
      s_tot = jnp.broadcast_to(jnp.sum(sv), (_L,))

      def cmax_body(c, car):
        c1, c2, c3, c4 = car
        v = cmax[c]
        a1 = jnp.maximum(c1, v)
        w = jnp.minimum(c1, v)
        a2 = jnp.maximum(c2, w)
        w = jnp.minimum(c2, w)
        a3 = jnp.maximum(c3, w)
        w = jnp.minimum(c3, w)
        a4 = jnp.maximum(c4, w)
        return a1, a2, a3, a4

      ninf = jnp.float32(_NEG)
      _, _, _, c4 = lax.fori_loop(0, n_chunks, cmax_body,
                                  (ninf, ninf, ninf, ninf))

      def collect_body(c, k):
        hit = cmax[c] >= c4

        @pl.when(hit)
        def _():
          cand[k] = c

        return k + jnp.where(hit, 1, 0)

      nc = lax.fori_loop(0, n_chunks, collect_body, jnp.int32(0))

      def rep_body(i, car):
        ta, tb = car
        c = cand[i]
        pltpu.sync_copy(x_hbm.at[pl.ds(base + c * chunk, chunk)], bufs[0])

        def rep_inner_body(j, car2):
          ta2, tb2 = car2
          off = j * (rep_unroll * _L)
          for u in range(rep_unroll):
            x = bufs[0][pl.ds(off + u * _L, _L)]
            if u % 2 == 0:
              ta2 = _insert4(*ta2, x)
            else:
              tb2 = _insert4(*tb2, x)
          return ta2, tb2

        return lax.fori_loop(0, rep_inner, rep_inner_body, (ta, tb),
                             unroll=False)

      t4 = ((neg_v,) * 4, (neg_v,) * 4)
      ta, tb = lax.fori_loop(0, nc, rep_body, t4)
      t0, t1, t2, t3 = _merge4(ta, tb)

      gv = neg_v
      for kk in range(4):
        gk = jnp.broadcast_to(jnp.max(t0), (_L,))
        eq = t0 == gk
        first = ii == plsc.all_reduce_ffs(eq)
        t0 = jnp.where(first, t1, t0)
        t1 = jnp.where(first, t2, t1)
        t2 = jnp.where(first, t3, t2)
        t3 = jnp.where(first, neg_v, t3)
        gv = jnp.where(ii == kk, gk, gv)

      res = jnp.where(ii == 4, mv, gv)
      res = jnp.where(ii == 5, s_tot, res)
      outb[...] = jnp.where(ii < 6, res, 0.0)
      pltpu.sync_copy(outb, out_hbm.at[row])

  return sc_kernel


def _tc_stream_body(x_ref, t0_ref, t1_ref, t2_ref, t3_ref, s_ref):
  first = pl.program_id(0) == 0
  x0 = x_ref[:, 0:128]
  ninf = jnp.full_like(x0, _NEG)
  t0 = jnp.where(first, ninf, t0_ref[...])
  t1 = jnp.where(first, ninf, t1_ref[...])
  t2 = jnp.where(first, ninf, t2_ref[...])
  t3 = jnp.where(first, ninf, t3_ref[...])
  s = jnp.where(first, 0.0, s_ref[...])
  bref = jnp.where(first, x0, t0)
  acc = jnp.zeros_like(x0)
  for i in range(_TC_CH // 128):
    xs = x_ref[:, 128 * i:128 * (i + 1)]
    t0, t1, t2, t3 = _insert4(t0, t1, t2, t3, xs)
    acc = acc + jnp.exp(xs - bref)
  s = (s + acc) * jnp.exp(bref - t0)
  t0_ref[...] = t0
  t1_ref[...] = t1
  t2_ref[...] = t2
  t3_ref[...] = t3
  s_ref[...] = s


def _tc_stream(x):
  part = jax.ShapeDtypeStruct((_B, 128), jnp.float32)
  return pl.pallas_call(
      _tc_stream_body,
      grid=(_TC_NB,),
      in_specs=[pl.BlockSpec((_B, _TC_CH), lambda b: (0, b))],
      out_specs=[pl.BlockSpec((_B, 128), lambda b: (0, 0))] * 5,
      out_shape=[part] * 5,
  )(x)


def _finalize_body(t0_ref, t1_ref, t2_ref, t3_ref, s_ref, sc_ref, head_ref,
                   out_ref):
  t0 = t0_ref[...]
  sc = sc_ref[...]
  m_sc = sc[:, 4:5]
  s_sc = sc[:, 5:6]
  mrow = jnp.maximum(jnp.max(t0, axis=1, keepdims=True), m_sc)
  stot = (jnp.sum(s_ref[...] * jnp.exp(t0 - mrow), axis=1, keepdims=True)
          + s_sc * jnp.exp(m_sc - mrow))
  lse = mrow + jnp.log(stot)

  sc4 = jnp.concatenate(
      [sc[:, 0:4], jnp.full((_B, 124), _NEG, jnp.float32)], axis=1)
  cat = jnp.concatenate([t0, t1_ref[...], t2_ref[...], t3_ref[...], sc4],
                        axis=1)
  idx = lax.broadcasted_iota(jnp.int32, cat.shape, 1)
  big = jnp.int32(1 << 30)
  gs = []
  for _ in range(4):
    gk = jnp.max(cat, axis=1, keepdims=True)
    eq = cat == gk
    fidx = jnp.min(jnp.where(eq, idx, big), axis=1, keepdims=True)
    cat = jnp.where(idx == fidx, _NEG, cat)
    gs.append(gk)

  thr = mrow + _LOG01
  cnt = ((gs[0] >= thr).astype(jnp.int32) + (gs[1] >= thr).astype(jnp.int32)
         + (gs[2] >= thr).astype(jnp.int32)
         + (gs[3] >= thr).astype(jnp.int32))

  head = head_ref[...]
  hidx = lax.broadcasted_iota(jnp.int32, head.shape, 1)
  fm = head < thr
  fbs = []
  for _ in range(3):
    fidx = jnp.min(jnp.where(fm, hidx, big), axis=1, keepdims=True)
    fbs.append(jnp.sum(jnp.where(hidx == fidx, head, 0.0), axis=1,
                       keepdims=True))
    fm = fm & (hidx != fidx)

  chosen = []
  for k in range(4):
    fb = fbs[2]
    if k >= 1:
      fb = jnp.where(cnt == k - 1, fbs[1], fb)
    fb = jnp.where(cnt == k, fbs[0], fb)
    chosen.append(jnp.where(cnt > k, gs[k], fb))

  c0, c1, c2, c3 = chosen
  a, b = jnp.maximum(c0, c1), jnp.minimum(c0, c1)
  c, d = jnp.maximum(c2, c3), jnp.minimum(c2, c3)
  c0, c2 = jnp.maximum(a, c), jnp.minimum(a, c)
  c1, c3 = jnp.maximum(b, d), jnp.minimum(b, d)
  c1, c2 = jnp.maximum(c1, c2), jnp.minimum(c1, c2)

  col = lax.broadcasted_iota(jnp.int32, (_B, 128), 1)
  outv = jnp.where(col == 0, c0, 0.0)
  outv = jnp.where(col == 1, c1, outv)
  outv = jnp.where(col == 2, c2, outv)
  outv = jnp.where(col == 3, c3, outv)
  out_ref[...] = jnp.where(col < 4, outv - lse, 0.0)


def _finalize(parts, sc_out, x):
  full = pl.BlockSpec((_B, 128), lambda i: (0, 0))
  return pl.pallas_call(
      _finalize_body,
      grid=(1,),
      in_specs=[full] * 5 + [pl.BlockSpec((_B, _L), lambda i: (0, 0)), full],
      out_specs=full,
      out_shape=jax.ShapeDtypeStruct((_B, 128), jnp.float32),
  )(*parts, sc_out, x)


@jax.jit
def kernel(logits, scores, beams):
  del beams
  sc = _build_sc(_V, _B, _V_TC, _V_SC, n_chunks=10, unroll=21, rep_unroll=5,
                 nbuf=10)
  sc_out = sc(logits.reshape(-1))
  parts = _tc_stream(logits)
  out = _finalize(parts, sc_out, logits)
  return out[:, :4].reshape(-1) + jnp.repeat(scores, 4)


# speedup vs baseline: 21.0852x; 21.0852x over previous
import math

import jax
import jax.numpy as jnp
from jax import lax
from jax.experimental import pallas as pl

_LOG01 = math.log(0.1)
_NEG = float("-inf")

_B = 64
_V = 1000000
_CH = 3200
_NB = 312
_TAIL = _V - _NB * _CH


def _insert4(t0, t1, t2, t3, x):
  m0 = jnp.maximum(t0, x)
  w = jnp.minimum(t0, x)
  m1 = jnp.maximum(t1, w)
  w = jnp.minimum(t1, w)
  m2 = jnp.maximum(t2, w)
  w = jnp.minimum(t2, w)
  m3 = jnp.maximum(t3, w)
  return m0, m1, m2, m3


def _stream_body(x_ref, t0_ref, t1_ref, t2_ref, t3_ref, s_ref):
  first = pl.program_id(0) == 0
  x0 = x_ref[:, 0:128]
  ninf = jnp.full_like(x0, _NEG)
  t0 = jnp.where(first, ninf, t0_ref[...])
  t1 = jnp.where(first, ninf, t1_ref[...])
  t2 = jnp.where(first, ninf, t2_ref[...])
  t3 = jnp.where(first, ninf, t3_ref[...])
  s = jnp.where(first, 0.0, s_ref[...])
  bref = jnp.where(first, x0, t0)
  acc = jnp.zeros_like(x0)
  for i in range(_CH // 128):
    xs = x_ref[:, 128 * i:128 * (i + 1)]
    t0, t1, t2, t3 = _insert4(t0, t1, t2, t3, xs)
    acc = acc + jnp.exp(xs - bref)
  s = (s + acc) * jnp.exp(bref - t0)
  t0_ref[...] = t0
  t1_ref[...] = t1
  t2_ref[...] = t2
  t3_ref[...] = t3
  s_ref[...] = s


def _stream(x):
  part = jax.ShapeDtypeStruct((_B, 128), jnp.float32)
  return pl.pallas_call(
      _stream_body,
      grid=(_NB,),
      in_specs=[pl.BlockSpec((_B, _CH), lambda b: (0, b))],
      out_specs=[pl.BlockSpec((_B, 128), lambda b: (0, 0))] * 5,
      out_shape=[part] * 5,
  )(x)


def _finalize_body(t0_ref, t1_ref, t2_ref, t3_ref, s_ref, head_ref, tail_ref,
                   out_ref):
  t0 = t0_ref[...]
  t1 = t1_ref[...]
  t2 = t2_ref[...]
  t3 = t3_ref[...]
  tref = t0

  acc = jnp.zeros_like(t0)
  nslice = _TAIL // 128
  for i in range(nslice):
    xs = tail_ref[:, 128 * i:128 * (i + 1)]
    t0, t1, t2, t3 = _insert4(t0, t1, t2, t3, xs)
    acc = acc + jnp.exp(xs - tref)
  rem = _TAIL - nslice * 128
  if rem:
    xr = tail_ref[:, nslice * 128:_TAIL]
    xs = jnp.concatenate(
        [xr, jnp.full((_B, 128 - rem), _NEG, jnp.float32)], axis=1)
    t0, t1, t2, t3 = _insert4(t0, t1, t2, t3, xs)
    acc = acc + jnp.exp(xs - tref)

  mrow = jnp.max(t0, axis=1, keepdims=True)
  stot = jnp.sum((s_ref[...] + acc) * jnp.exp(tref - mrow), axis=1,
                 keepdims=True)
  lse = mrow + jnp.log(stot)

  cat = jnp.concatenate([t0, t1, t2, t3], axis=1)
  idx = lax.broadcasted_iota(jnp.int32, cat.shape, 1)
  big = jnp.int32(1 << 30)
  gs = []
  for _ in range(4):
    gk = jnp.max(cat, axis=1, keepdims=True)
    eq = cat == gk
    fidx = jnp.min(jnp.where(eq, idx, big), axis=1, keepdims=True)
    cat = jnp.where(idx == fidx, _NEG, cat)
    gs.append(gk)

  thr = mrow + _LOG01
  cnt = ((gs[0] >= thr).astype(jnp.int32) + (gs[1] >= thr).astype(jnp.int32)
         + (gs[2] >= thr).astype(jnp.int32)
         + (gs[3] >= thr).astype(jnp.int32))

  head = head_ref[...]
  hidx = lax.broadcasted_iota(jnp.int32, head.shape, 1)
  fm = head < thr
  fbs = []
  for _ in range(3):
    fidx = jnp.min(jnp.where(fm, hidx, big), axis=1, keepdims=True)
    fbs.append(jnp.sum(jnp.where(hidx == fidx, head, 0.0), axis=1,
                       keepdims=True))
    fm = fm & (hidx != fidx)

  chosen = []
  for k in range(4):
    fb = fbs[2]
    if k >= 1:
      fb = jnp.where(cnt == k - 1, fbs[1], fb)
    fb = jnp.where(cnt == k, fbs[0], fb)
    chosen.append(jnp.where(cnt > k, gs[k], fb))

  c0, c1, c2, c3 = chosen
  a, b = jnp.maximum(c0, c1), jnp.minimum(c0, c1)
  c, d = jnp.maximum(c2, c3), jnp.minimum(c2, c3)
  c0, c2 = jnp.maximum(a, c), jnp.minimum(a, c)
  c1, c3 = jnp.maximum(b, d), jnp.minimum(b, d)
  c1, c2 = jnp.maximum(c1, c2), jnp.minimum(c1, c2)

  col = lax.broadcasted_iota(jnp.int32, (_B, 128), 1)
  outv = jnp.where(col == 0, c0, 0.0)
  outv = jnp.where(col == 1, c1, outv)
  outv = jnp.where(col == 2, c2, outv)
  outv = jnp.where(col == 3, c3, outv)
  out_ref[...] = jnp.where(col < 4, outv - lse, 0.0)


def _finalize(parts, x):
  full = pl.BlockSpec((_B, 128), lambda i: (0, 0))
  tail = pl.BlockSpec((_B, 6400), lambda i: (0, (_NB * _CH) // 6400))
  return pl.pallas_call(
      _finalize_body,
      grid=(1,),
      in_specs=[full] * 5 + [full, tail],
      out_specs=full,
      out_shape=jax.ShapeDtypeStruct((_B, 128), jnp.float32),
  )(*parts, x, x)


@jax.jit
def kernel(logits, scores, beams):
  del beams
  parts = _stream(logits)
  out = _finalize(parts, logits)
  return out[:, :4].reshape(-1) + jnp.repeat(scores, 4)


# speedup vs baseline: 38.2789x; 1.8154x over previous
import math

import jax
import jax.numpy as jnp
from jax import lax
from jax.experimental import pallas as pl

_LOG01 = math.log(0.1)
_NEG = float("-inf")

_B = 64
_V = 1000000
_CH = 12800
_NB = 78
_TAIL = _V - _NB * _CH


def _insert4(t0, t1, t2, t3, x):
  m0 = jnp.maximum(t0, x)
  w = jnp.minimum(t0, x)
  m1 = jnp.maximum(t1, w)
  w = jnp.minimum(t1, w)
  m2 = jnp.maximum(t2, w)
  w = jnp.minimum(t2, w)
  m3 = jnp.maximum(t3, w)
  return m0, m1, m2, m3


def _stream_body(x_ref, t0_ref, t1_ref, t2_ref, t3_ref, s_ref):
  first = pl.program_id(0) == 0
  x0 = x_ref[:, 0:128]
  ninf = jnp.full_like(x0, _NEG)
  t0 = jnp.where(first, ninf, t0_ref[...])
  t1 = jnp.where(first, ninf, t1_ref[...])
  t2 = jnp.where(first, ninf, t2_ref[...])
  t3 = jnp.where(first, ninf, t3_ref[...])
  s = jnp.where(first, 0.0, s_ref[...])
  bref = jnp.where(first, x0, t0)
  acc = jnp.zeros_like(x0)
  for i in range(_CH // 128):
    xs = x_ref[:, 128 * i:128 * (i + 1)]
    t0, t1, t2, t3 = _insert4(t0, t1, t2, t3, xs)
    acc = acc + jnp.exp(xs - bref)
  s = (s + acc) * jnp.exp(bref - t0)
  t0_ref[...] = t0
  t1_ref[...] = t1
  t2_ref[...] = t2
  t3_ref[...] = t3
  s_ref[...] = s


def _stream(x):
  part = jax.ShapeDtypeStruct((_B, 128), jnp.float32)
  return pl.pallas_call(
      _stream_body,
      grid=(_NB,),
      in_specs=[pl.BlockSpec((_B, _CH), lambda b: (0, b))],
      out_specs=[pl.BlockSpec((_B, 128), lambda b: (0, 0))] * 5,
      out_shape=[part] * 5,
  )(x)


def _finalize_body(t0_ref, t1_ref, t2_ref, t3_ref, s_ref, head_ref, tail_ref,
                   out_ref):
  t0 = t0_ref[...]
  t1 = t1_ref[...]
  t2 = t2_ref[...]
  t3 = t3_ref[...]
  tref = t0

  acc = jnp.zeros_like(t0)
  nslice = _TAIL // 128
  for i in range(nslice):
    xs = tail_ref[:, 128 * i:128 * (i + 1)]
    t0, t1, t2, t3 = _insert4(t0, t1, t2, t3, xs)
    acc = acc + jnp.exp(xs - tref)
  rem = _TAIL - nslice * 128
  if rem:
    xr = tail_ref[:, nslice * 128:_TAIL]
    xs = jnp.concatenate(
        [xr, jnp.full((_B, 128 - rem), _NEG, jnp.float32)], axis=1)
    t0, t1, t2, t3 = _insert4(t0, t1, t2, t3, xs)
    acc = acc + jnp.exp(xs - tref)

  mrow = jnp.max(t0, axis=1, keepdims=True)
  stot = jnp.sum((s_ref[...] + acc) * jnp.exp(tref - mrow), axis=1,
                 keepdims=True)
  lse = mrow + jnp.log(stot)

  cat = jnp.concatenate([t0, t1, t2, t3], axis=1)
  idx = lax.broadcasted_iota(jnp.int32, cat.shape, 1)
  big = jnp.int32(1 << 30)
  gs = []
  for _ in range(4):
    gk = jnp.max(cat, axis=1, keepdims=True)
    eq = cat == gk
    fidx = jnp.min(jnp.where(eq, idx, big), axis=1, keepdims=True)
    cat = jnp.where(idx == fidx, _NEG, cat)
    gs.append(gk)

  thr = mrow + _LOG01
  cnt = ((gs[0] >= thr).astype(jnp.int32) + (gs[1] >= thr).astype(jnp.int32)
         + (gs[2] >= thr).astype(jnp.int32)
         + (gs[3] >= thr).astype(jnp.int32))

  head = head_ref[...]
  hidx = lax.broadcasted_iota(jnp.int32, head.shape, 1)
  fm = head < thr
  fbs = []
  for _ in range(3):
    fidx = jnp.min(jnp.where(fm, hidx, big), axis=1, keepdims=True)
    fbs.append(jnp.sum(jnp.where(hidx == fidx, head, 0.0), axis=1,
                       keepdims=True))
    fm = fm & (hidx != fidx)

  chosen = []
  for k in range(4):
    fb = fbs[2]
    if k >= 1:
      fb = jnp.where(cnt == k - 1, fbs[1], fb)
    fb = jnp.where(cnt == k, fbs[0], fb)
    chosen.append(jnp.where(cnt > k, gs[k], fb))

  c0, c1, c2, c3 = chosen
  a, b = jnp.maximum(c0, c1), jnp.minimum(c0, c1)
  c, d = jnp.maximum(c2, c3), jnp.minimum(c2, c3)
  c0, c2 = jnp.maximum(a, c), jnp.minimum(a, c)
  c1, c3 = jnp.maximum(b, d), jnp.minimum(b, d)
  c1, c2 = jnp.maximum(c1, c2), jnp.minimum(c1, c2)

  col = lax.broadcasted_iota(jnp.int32, (_B, 128), 1)
  outv = jnp.where(col == 0, c0, 0.0)
  outv = jnp.where(col == 1, c1, outv)
  outv = jnp.where(col == 2, c2, outv)
  outv = jnp.where(col == 3, c3, outv)
  out_ref[...] = jnp.where(col < 4, outv - lse, 0.0)


def _finalize(parts, x):
  full = pl.BlockSpec((_B, 128), lambda i: (0, 0))
  tail = pl.BlockSpec((_B, 6400), lambda i: (0, (_NB * _CH) // 6400))
  return pl.pallas_call(
      _finalize_body,
      grid=(1,),
      in_specs=[full] * 5 + [full, tail],
      out_specs=full,
      out_shape=jax.ShapeDtypeStruct((_B, 128), jnp.float32),
  )(*parts, x, x)


@jax.jit
def kernel(logits, scores, beams):
  del beams
  parts = _stream(logits)
  out = _finalize(parts, logits)
  return out[:, :4].reshape(-1) + jnp.repeat(scores, 4)


# speedup vs baseline: 44.3058x; 1.1574x over previous
import math

import jax
import jax.numpy as jnp
from jax import lax
from jax.experimental import pallas as pl

_LOG01 = math.log(0.1)
_NEG = float("-inf")

_B = 64
_V = 1000000
_CH = 25600
_NB = 39
_TAIL = _V - _NB * _CH


def _insert4(t0, t1, t2, t3, x):
  m0 = jnp.maximum(t0, x)
  w = jnp.minimum(t0, x)
  m1 = jnp.maximum(t1, w)
  w = jnp.minimum(t1, w)
  m2 = jnp.maximum(t2, w)
  w = jnp.minimum(t2, w)
  m3 = jnp.maximum(t3, w)
  return m0, m1, m2, m3


def _stream_body(x_ref, t0_ref, t1_ref, t2_ref, t3_ref, s_ref):
  first = pl.program_id(0) == 0
  x0 = x_ref[:, 0:128]
  ninf = jnp.full_like(x0, _NEG)
  t0 = jnp.where(first, ninf, t0_ref[...])
  t1 = jnp.where(first, ninf, t1_ref[...])
  t2 = jnp.where(first, ninf, t2_ref[...])
  t3 = jnp.where(first, ninf, t3_ref[...])
  s = jnp.where(first, 0.0, s_ref[...])
  bref = jnp.where(first, x0, t0)
  acc = jnp.zeros_like(x0)
  for i in range(_CH // 128):
    xs = x_ref[:, 128 * i:128 * (i + 1)]
    t0, t1, t2, t3 = _insert4(t0, t1, t2, t3, xs)
    acc = acc + jnp.exp(xs - bref)
  s = (s + acc) * jnp.exp(bref - t0)
  t0_ref[...] = t0
  t1_ref[...] = t1
  t2_ref[...] = t2
  t3_ref[...] = t3
  s_ref[...] = s


def _stream(x):
  part = jax.ShapeDtypeStruct((_B, 128), jnp.float32)
  return pl.pallas_call(
      _stream_body,
      grid=(_NB,),
      in_specs=[pl.BlockSpec((_B, _CH), lambda b: (0, b))],
      out_specs=[pl.BlockSpec((_B, 128), lambda b: (0, 0))] * 5,
      out_shape=[part] * 5,
  )(x)


def _finalize_body(t0_ref, t1_ref, t2_ref, t3_ref, s_ref, head_ref, tail_ref,
                   out_ref):
  t0 = t0_ref[...]
  t1 = t1_ref[...]
  t2 = t2_ref[...]
  t3 = t3_ref[...]
  tref = t0

  acc = jnp.zeros_like(t0)
  nslice = _TAIL // 128
  for i in range(nslice):
    xs = tail_ref[:, 128 * i:128 * (i + 1)]
    t0, t1, t2, t3 = _insert4(t0, t1, t2, t3, xs)
    acc = acc + jnp.exp(xs - tref)
  rem = _TAIL - nslice * 128
  if rem:
    xr = tail_ref[:, nslice * 128:_TAIL]
    xs = jnp.concatenate(
        [xr, jnp.full((_B, 128 - rem), _NEG, jnp.float32)], axis=1)
    t0, t1, t2, t3 = _insert4(t0, t1, t2, t3, xs)
    acc = acc + jnp.exp(xs - tref)

  mrow = jnp.max(t0, axis=1, keepdims=True)
  stot = jnp.sum((s_ref[...] + acc) * jnp.exp(tref - mrow), axis=1,
                 keepdims=True)
  lse = mrow + jnp.log(stot)

  cat = jnp.concatenate([t0, t1, t2, t3], axis=1)
  idx = lax.broadcasted_iota(jnp.int32, cat.shape, 1)
  big = jnp.int32(1 << 30)
  gs = []
  for _ in range(4):
    gk = jnp.max(cat, axis=1, keepdims=True)
    eq = cat == gk
    fidx = jnp.min(jnp.where(eq, idx, big), axis=1, keepdims=True)
    cat = jnp.where(idx == fidx, _NEG, cat)
    gs.append(gk)

  thr = mrow + _LOG01
  cnt = ((gs[0] >= thr).astype(jnp.int32) + (gs[1] >= thr).astype(jnp.int32)
         + (gs[2] >= thr).astype(jnp.int32)
         + (gs[3] >= thr).astype(jnp.int32))

  head = head_ref[...]
  hidx = lax.broadcasted_iota(jnp.int32, head.shape, 1)
  fm = head < thr
  fbs = []
  for _ in range(3):
    fidx = jnp.min(jnp.where(fm, hidx, big), axis=1, keepdims=True)
    fbs.append(jnp.sum(jnp.where(hidx == fidx, head, 0.0), axis=1,
                       keepdims=True))
    fm = fm & (hidx != fidx)

  chosen = []
  for k in range(4):
    fb = fbs[2]
    if k >= 1:
      fb = jnp.where(cnt == k - 1, fbs[1], fb)
    fb = jnp.where(cnt == k, fbs[0], fb)
    chosen.append(jnp.where(cnt > k, gs[k], fb))

  c0, c1, c2, c3 = chosen
  a, b = jnp.maximum(c0, c1), jnp.minimum(c0, c1)
  c, d = jnp.maximum(c2, c3), jnp.minimum(c2, c3)
  c0, c2 = jnp.maximum(a, c), jnp.minimum(a, c)
  c1, c3 = jnp.maximum(b, d), jnp.minimum(b, d)
  c1, c2 = jnp.maximum(c1, c2), jnp.minimum(c1, c2)

  col = lax.broadcasted_iota(jnp.int32, (_B, 128), 1)
  outv = jnp.where(col == 0, c0, 0.0)
  outv = jnp.where(col == 1, c1, outv)
  outv = jnp.where(col == 2, c2, outv)
  outv = jnp.where(col == 3, c3, outv)
  out_ref[...] = jnp.where(col < 4, outv - lse, 0.0)


def _finalize(parts, x):
  full = pl.BlockSpec((_B, 128), lambda i: (0, 0))
  tail = pl.BlockSpec((_B, 6400), lambda i: (0, (_NB * _CH) // 6400))
  return pl.pallas_call(
      _finalize_body,
      grid=(1,),
      in_specs=[full] * 5 + [full, tail],
      out_specs=full,
      out_shape=jax.ShapeDtypeStruct((_B, 128), jnp.float32),
  )(*parts, x, x)


@jax.jit
def kernel(logits, scores, beams):
  del beams
  parts = _stream(logits)
  out = _finalize(parts, logits)
  return out[:, :4].reshape(-1) + jnp.repeat(scores, 4)


# speedup vs baseline: 45.4927x; 1.0268x over previous
import math

import jax
import jax.numpy as jnp
from jax import lax
from jax.experimental import pallas as pl

_LOG01 = math.log(0.1)
_NEG = float("-inf")

_B = 64
_V = 1000000
_CH = 49920
_NB = 20
_TAIL = _V - _NB * _CH


def _insert4(t0, t1, t2, t3, x):
  m0 = jnp.maximum(t0, x)
  w = jnp.minimum(t0, x)
  m1 = jnp.maximum(t1, w)
  w = jnp.minimum(t1, w)
  m2 = jnp.maximum(t2, w)
  w = jnp.minimum(t2, w)
  m3 = jnp.maximum(t3, w)
  return m0, m1, m2, m3


def _stream_body(x_ref, t0_ref, t1_ref, t2_ref, t3_ref, s_ref):
  first = pl.program_id(0) == 0
  x0 = x_ref[:, 0:128]
  ninf = jnp.full_like(x0, _NEG)
  t0 = jnp.where(first, ninf, t0_ref[...])
  t1 = jnp.where(first, ninf, t1_ref[...])
  t2 = jnp.where(first, ninf, t2_ref[...])
  t3 = jnp.where(first, ninf, t3_ref[...])
  s = jnp.where(first, 0.0, s_ref[...])
  bref = jnp.where(first, x0, t0)
  acc = jnp.zeros_like(x0)
  for i in range(_CH // 128):
    xs = x_ref[:, 128 * i:128 * (i + 1)]
    t0, t1, t2, t3 = _insert4(t0, t1, t2, t3, xs)
    acc = acc + jnp.exp(xs - bref)
  s = (s + acc) * jnp.exp(bref - t0)
  t0_ref[...] = t0
  t1_ref[...] = t1
  t2_ref[...] = t2
  t3_ref[...] = t3
  s_ref[...] = s


def _stream(x):
  part = jax.ShapeDtypeStruct((_B, 128), jnp.float32)
  return pl.pallas_call(
      _stream_body,
      grid=(_NB,),
      in_specs=[pl.BlockSpec((_B, _CH), lambda b: (0, b))],
      out_specs=[pl.BlockSpec((_B, 128), lambda b: (0, 0))] * 5,
      out_shape=[part] * 5,
  )(x)


def _finalize_body(t0_ref, t1_ref, t2_ref, t3_ref, s_ref, head_ref, tail_ref,
                   out_ref):
  t0 = t0_ref[...]
  t1 = t1_ref[...]
  t2 = t2_ref[...]
  t3 = t3_ref[...]
  tref = t0

  acc = jnp.zeros_like(t0)
  nslice = _TAIL // 128
  for i in range(nslice):
    xs = tail_ref[:, 128 * i:128 * (i + 1)]
    t0, t1, t2, t3 = _insert4(t0, t1, t2, t3, xs)
    acc = acc + jnp.exp(xs - tref)
  rem = _TAIL - nslice * 128
  if rem:
    xr = tail_ref[:, nslice * 128:_TAIL]
    xs = jnp.concatenate(
        [xr, jnp.full((_B, 128 - rem), _NEG, jnp.float32)], axis=1)
    t0, t1, t2, t3 = _insert4(t0, t1, t2, t3, xs)
    acc = acc + jnp.exp(xs - tref)

  mrow = jnp.max(t0, axis=1, keepdims=True)
  stot = jnp.sum((s_ref[...] + acc) * jnp.exp(tref - mrow), axis=1,
                 keepdims=True)
  lse = mrow + jnp.log(stot)

  cat = jnp.concatenate([t0, t1, t2, t3], axis=1)
  idx = lax.broadcasted_iota(jnp.int32, cat.shape, 1)
  big = jnp.int32(1 << 30)
  gs = []
  for _ in range(4):
    gk = jnp.max(cat, axis=1, keepdims=True)
    eq = cat == gk
    fidx = jnp.min(jnp.where(eq, idx, big), axis=1, keepdims=True)
    cat = jnp.where(idx == fidx, _NEG, cat)
    gs.append(gk)

  thr = mrow + _LOG01
  cnt = ((gs[0] >= thr).astype(jnp.int32) + (gs[1] >= thr).astype(jnp.int32)
         + (gs[2] >= thr).astype(jnp.int32)
         + (gs[3] >= thr).astype(jnp.int32))

  head = head_ref[...]
  hidx = lax.broadcasted_iota(jnp.int32, head.shape, 1)
  fm = head < thr
  fbs = []
  for _ in range(3):
    fidx = jnp.min(jnp.where(fm, hidx, big), axis=1, keepdims=True)
    fbs.append(jnp.sum(jnp.where(hidx == fidx, head, 0.0), axis=1,
                       keepdims=True))
    fm = fm & (hidx != fidx)

  chosen = []
  for k in range(4):
    fb = fbs[2]
    if k >= 1:
      fb = jnp.where(cnt == k - 1, fbs[1], fb)
    fb = jnp.where(cnt == k, fbs[0], fb)
    chosen.append(jnp.where(cnt > k, gs[k], fb))

  c0, c1, c2, c3 = chosen
  a, b = jnp.maximum(c0, c1), jnp.minimum(c0, c1)
  c, d = jnp.maximum(c2, c3), jnp.minimum(c2, c3)
  c0, c2 = jnp.maximum(a, c), jnp.minimum(a, c)
  c1, c3 = jnp.maximum(b, d), jnp.minimum(b, d)
  c1, c2 = jnp.maximum(c1, c2), jnp.minimum(c1, c2)

  col = lax.broadcasted_iota(jnp.int32, (_B, 128), 1)
  outv = jnp.where(col == 0, c0, 0.0)
  outv = jnp.where(col == 1, c1, outv)
  outv = jnp.where(col == 2, c2, outv)
  outv = jnp.where(col == 3, c3, outv)
  out_ref[...] = jnp.where(col < 4, outv - lse, 0.0)


def _finalize(parts, x):
  full = pl.BlockSpec((_B, 128), lambda i: (0, 0))
  tail = pl.BlockSpec((_B, 6400), lambda i: (0, (_NB * _CH) // 6400))
  return pl.pallas_call(
      _finalize_body,
      grid=(1,),
      in_specs=[full] * 5 + [full, tail],
      out_specs=full,
      out_shape=jax.ShapeDtypeStruct((_B, 128), jnp.float32),
  )(*parts, x, x)


@jax.jit
def kernel(logits, scores, beams):
  del beams
  parts = _stream(logits)
  out = _finalize(parts, logits)
  return out[:, :4].reshape(-1) + jnp.repeat(scores, 4)
